# Initial kernel scaffold; baseline (speedup 1.0000x reference)
import functools

import jax
import jax.numpy as jnp
from jax import lax
from jax.experimental import pallas as pl
from jax.experimental.pallas import tpu as pltpu
from jax.experimental.pallas import tpu_sc as plsc

NC = 2
NS = 16
LANES = 16


def _sc_sums(tokens3, table):
    B, _, HALF = tokens3.shape
    S = 2 * HALF
    _, D = table.shape
    NW = NC * NS
    BPW = B // NW
    NVREG = D // LANES

    mesh = plsc.VectorSubcoreMesh(core_axis_name="c", subcore_axis_name="s")

    @functools.partial(
        pl.kernel,
        out_type=jax.ShapeDtypeStruct((B, D), jnp.float32),
        mesh=mesh,
        scratch_types=[
            pltpu.VMEM((BPW, 2, HALF), jnp.int32),
            pltpu.VMEM((2, S, D), jnp.float32),
            pltpu.VMEM((BPW, D), jnp.float32),
            pltpu.SemaphoreType.DMA((2,)),
        ],
    )
    def sc_kernel(tokens_hbm, table_hbm, out_hbm, idx_v, rows_v, out_v, sems):
        wid = lax.axis_index("s") * NC + lax.axis_index("c")
        base = wid * BPW
        pltpu.sync_copy(tokens_hbm.at[pl.ds(base, BPW)], idx_v)

        def issue(r, buf):
            for j in range(2):
                pltpu.async_copy(
                    table_hbm.at[idx_v.at[r, j]],
                    rows_v.at[buf, pl.ds(j * HALF, HALF)],
                    sems.at[buf],
                )

        def drain(buf):
            for j in range(2):
                pltpu.make_async_copy(
                    table_hbm.at[idx_v.at[0, j]],
                    rows_v.at[buf, pl.ds(j * HALF, HALF)],
                    sems.at[buf],
                ).wait()

        issue(0, 0)

        def pair_body(i, carry):
            for b in range(2):
                r = 2 * i + b

                @pl.when(r + 1 < BPW)
                def _():
                    issue(r + 1, 1 - b)

                drain(b)

                def tok_body(t, accs):
                    return tuple(
                        a + rows_v[b, t, pl.ds(k * LANES, LANES)]
                        for k, a in enumerate(accs)
                    )

                accs = lax.fori_loop(
                    0, S, tok_body,
                    tuple(jnp.zeros((LANES,), jnp.float32) for _ in range(NVREG)),
                )
                for k, a in enumerate(accs):
                    out_v[r, pl.ds(k * LANES, LANES)] = a
            return carry

        lax.fori_loop(0, BPW // 2, pair_body, 0)
        pltpu.sync_copy(out_v, out_hbm.at[pl.ds(base, BPW)])

    return sc_kernel(tokens3, table)


def _tc_head(sums, tokens, row0, W1, b1, W2, b2, W3, b3, W4, b4):
    B, D = sums.shape
    _, S = tokens.shape
    H = W1.shape[1]
    O = W2.shape[1]
    BLK = 512

    def body(sums_ref, tok_ref, row0_ref, W1r, b1r, W2r, b2r, W3r, b3r,
             W4r, b4r, loc_ref, scale_ref):
        tok = tok_ref[...]
        cnt = jnp.sum((tok != 0).astype(jnp.float32), axis=1, keepdims=True)
        npad = jnp.float32(S) - cnt
        s = sums_ref[...] - npad * row0_ref[...]
        avg = s / cnt
        h1 = jnp.maximum(
            jnp.dot(avg, W1r[...], preferred_element_type=jnp.float32)
            + b1r[...], 0.0)
        loc_ref[...] = (
            jnp.dot(h1, W2r[...], preferred_element_type=jnp.float32)
            + b2r[...])
        h2 = jnp.maximum(
            jnp.dot(avg, W3r[...], preferred_element_type=jnp.float32)
            + b3r[...], 0.0)
        scale_ref[...] = jnp.exp(
            jnp.dot(h2, W4r[...], preferred_element_type=jnp.float32)
            + b4r[...])

    grid = (B // BLK,)
    full = lambda shape: pl.BlockSpec(shape, lambda i: (0, 0))
    return pl.pallas_call(
        body,
        grid=grid,
        in_specs=[
            pl.BlockSpec((BLK, D), lambda i: (i, 0)),
            pl.BlockSpec((BLK, S), lambda i: (i, 0)),
            full((1, D)),
            full((D, H)), full((1, H)),
            full((H, O)), full((1, O)),
            full((D, H)), full((1, H)),
            full((H, O)), full((1, O)),
        ],
        out_specs=[
            pl.BlockSpec((BLK, O), lambda i: (i, 0)),
            pl.BlockSpec((BLK, O), lambda i: (i, 0)),
        ],
        out_shape=[
            jax.ShapeDtypeStruct((B, O), jnp.float32),
            jax.ShapeDtypeStruct((B, O), jnp.float32),
        ],
    )(sums, tokens, row0, W1, b1, W2, b2, W3, b3, W4, b4)


def kernel(tokens, table, W1, b1, W2, b2, W3, b3, W4, b4):
    B, S = tokens.shape
    sums = _sc_sums(tokens.reshape(B, 2, S // 2), table)
    loc, scale = _tc_head(
        sums, tokens, table[0:1],
        W1, b1.reshape(1, -1), W2, b2.reshape(1, -1),
        W3, b3.reshape(1, -1), W4, b4.reshape(1, -1))
    return (loc, scale)


# speedup vs baseline: 13.8794x; 13.8794x over previous
import functools

import jax
import jax.numpy as jnp
from jax import lax
from jax.experimental import pallas as pl
from jax.experimental.pallas import tpu as pltpu
from jax.experimental.pallas import tpu_sc as plsc

NC = 2
NS = 16
LANES = 16


def _sc_sums(tokens3, table):
    B, _, HALF = tokens3.shape
    S = 2 * HALF
    _, D = table.shape
    NW = NC * NS
    BPW = B // NW
    NVREG = D // LANES

    mesh = plsc.VectorSubcoreMesh(core_axis_name="c", subcore_axis_name="s")

    @functools.partial(
        pl.kernel,
        out_type=jax.ShapeDtypeStruct((B, D), jnp.float32),
        mesh=mesh,
        scratch_types=[
            pltpu.VMEM((BPW, 2, HALF), jnp.int32),
            pltpu.VMEM((2, S, D), jnp.float32),
            pltpu.VMEM((BPW, D), jnp.float32),
            pltpu.SemaphoreType.DMA((2,)),
        ],
        compiler_params=pltpu.CompilerParams(use_tc_tiling_on_sc=False),
    )
    def sc_kernel(tokens_hbm, table_hbm, out_hbm, idx_v, rows_v, out_v, sems):
        wid = lax.axis_index("s") * NC + lax.axis_index("c")
        base = wid * BPW
        pltpu.sync_copy(tokens_hbm.at[pl.ds(base, BPW)], idx_v)

        def issue(r, buf):
            for j in range(2):
                pltpu.async_copy(
                    table_hbm.at[idx_v.at[r, j]],
                    rows_v.at[buf, pl.ds(j * HALF, HALF)],
                    sems.at[buf],
                )

        def drain(buf):
            for j in range(2):
                pltpu.make_async_copy(
                    table_hbm.at[idx_v.at[0, j]],
                    rows_v.at[buf, pl.ds(j * HALF, HALF)],
                    sems.at[buf],
                ).wait()

        issue(0, 0)

        def pair_body(i, carry):
            for b in range(2):
                r = 2 * i + b

                @pl.when(r + 1 < BPW)
                def _():
                    issue(r + 1, 1 - b)

                drain(b)

                def tok_body(t, accs):
                    return tuple(
                        a + rows_v[b, t, pl.ds(k * LANES, LANES)]
                        for k, a in enumerate(accs)
                    )

                accs = lax.fori_loop(
                    0, S, tok_body,
                    tuple(jnp.zeros((LANES,), jnp.float32) for _ in range(NVREG)),
                )
                for k, a in enumerate(accs):
                    out_v[r, pl.ds(k * LANES, LANES)] = a
            return carry

        lax.fori_loop(0, BPW // 2, pair_body, 0)
        pltpu.sync_copy(out_v, out_hbm.at[pl.ds(base, BPW)])

    return sc_kernel(tokens3, table)


def _tc_head(sums, tokens, row0, W1, b1, W2, b2, W3, b3, W4, b4):
    B, D = sums.shape
    _, S = tokens.shape
    H = W1.shape[1]
    O = W2.shape[1]
    BLK = 512

    def body(sums_ref, tok_ref, row0_ref, W1r, b1r, W2r, b2r, W3r, b3r,
             W4r, b4r, loc_ref, scale_ref):
        tok = tok_ref[...]
        cnt = jnp.sum((tok != 0).astype(jnp.float32), axis=1, keepdims=True)
        npad = jnp.float32(S) - cnt
        s = sums_ref[...] - npad * row0_ref[...]
        avg = s / cnt
        h1 = jnp.maximum(
            jnp.dot(avg, W1r[...], preferred_element_type=jnp.float32)
            + b1r[...], 0.0)
        loc_ref[...] = (
            jnp.dot(h1, W2r[...], preferred_element_type=jnp.float32)
            + b2r[...])
        h2 = jnp.maximum(
            jnp.dot(avg, W3r[...], preferred_element_type=jnp.float32)
            + b3r[...], 0.0)
        scale_ref[...] = jnp.exp(
            jnp.dot(h2, W4r[...], preferred_element_type=jnp.float32)
            + b4r[...])

    grid = (B // BLK,)
    full = lambda shape: pl.BlockSpec(shape, lambda i: (0, 0))
    return pl.pallas_call(
        body,
        grid=grid,
        in_specs=[
            pl.BlockSpec((BLK, D), lambda i: (i, 0)),
            pl.BlockSpec((BLK, S), lambda i: (i, 0)),
            full((1, D)),
            full((D, H)), full((1, H)),
            full((H, O)), full((1, O)),
            full((D, H)), full((1, H)),
            full((H, O)), full((1, O)),
        ],
        out_specs=[
            pl.BlockSpec((BLK, O), lambda i: (i, 0)),
            pl.BlockSpec((BLK, O), lambda i: (i, 0)),
        ],
        out_shape=[
            jax.ShapeDtypeStruct((B, O), jnp.float32),
            jax.ShapeDtypeStruct((B, O), jnp.float32),
        ],
    )(sums, tokens, row0, W1, b1, W2, b2, W3, b3, W4, b4)


def kernel(tokens, table, W1, b1, W2, b2, W3, b3, W4, b4):
    B, S = tokens.shape
    sums = _sc_sums(tokens.reshape(B, 2, S // 2), table)
    loc, scale = _tc_head(
        sums, tokens, table[0:1],
        W1, b1.reshape(1, -1), W2, b2.reshape(1, -1),
        W3, b3.reshape(1, -1), W4, b4.reshape(1, -1))
    return (loc, scale)
